# baseline (speedup 1.0000x reference)
import jax
import jax.numpy as jnp
from jax.experimental import pallas as pl
from jax.experimental.pallas import tpu as pltpu

_BT = 512
_NBUF = 4


def _make_inner(tokens, model_dim, num_experts):
    def inner(x_hbm, w_ref, o_hbm):
        def body(x_blk, o_blk):
            o_blk[...] = jax.lax.dot_general(
                x_blk[...], w_ref[...],
                dimension_numbers=(((1,), (1,)), ((), ())),
                preferred_element_type=jnp.float32,
            )

        pipeline = pltpu.emit_pipeline(
            body,
            grid=(tokens // _BT,),
            in_specs=[
                pl.BlockSpec((_BT, model_dim), lambda i: (i, 0),
                             pipeline_mode=pl.Buffered(buffer_count=_NBUF)),
            ],
            out_specs=[
                pl.BlockSpec((_BT, num_experts), lambda i: (i, 0)),
            ],
        )
        pipeline(x_hbm, o_hbm)

    return inner


def kernel(x, wg_weight):
    tokens, model_dim = x.shape
    num_experts = wg_weight.shape[0]
    return pl.pallas_call(
        _make_inner(tokens, model_dim, num_experts),
        in_specs=[
            pl.BlockSpec(memory_space=pl.ANY),
            pl.BlockSpec(memory_space=pltpu.VMEM),
        ],
        out_specs=pl.BlockSpec(memory_space=pl.ANY),
        out_shape=jax.ShapeDtypeStruct((tokens, num_experts), jnp.float32),
    )(x, wg_weight)


# speedup vs baseline: 1.0132x; 1.0132x over previous
import jax
import jax.numpy as jnp
from jax.experimental import pallas as pl
from jax.experimental.pallas import tpu as pltpu

_BT = 512


def _gate_kernel(x_ref, w_ref, o_ref):
    o_ref[...] = jax.lax.dot_general(
        x_ref[...], w_ref[...],
        dimension_numbers=(((1,), (1,)), ((), ())),
        preferred_element_type=jnp.float32,
    )


def kernel(x, wg_weight):
    tokens, model_dim = x.shape
    num_experts = wg_weight.shape[0]
    return pl.pallas_call(
        _gate_kernel,
        grid=(tokens // _BT,),
        in_specs=[
            pl.BlockSpec((_BT, model_dim), lambda i: (i, 0)),
            pl.BlockSpec((num_experts, model_dim), lambda i: (0, 0)),
        ],
        out_specs=pl.BlockSpec((_BT, num_experts), lambda i: (i, 0)),
        out_shape=jax.ShapeDtypeStruct((tokens, num_experts), jnp.float32),
        compiler_params=pltpu.CompilerParams(
            dimension_semantics=("parallel",),
        ),
    )(x, wg_weight)
